# Initial kernel scaffold; baseline (speedup 1.0000x reference)
import jax
import jax.numpy as jnp
from jax.experimental import pallas as pl


def kernel(x, W1, g1, b1, W2, g2, b2, W3, g3, b3, W4, g4, b4, W5, g5, b5, W6, g6, b6, W7, bias7, g7, b7, W8, bias8):
    raise NotImplementedError("write your pallas kernel here")


# speedup vs baseline: 10.1557x; 10.1557x over previous
import functools

import jax
import jax.numpy as jnp
from jax import lax
from jax.experimental import pallas as pl
from jax.experimental.pallas import tpu as pltpu
from jax.experimental.pallas import tpu_sc as plsc

F32 = jnp.float32
BF16 = jnp.bfloat16
I32 = jnp.int32

B = 16
N = 2048
KNN = 20
CP = 128
EPS = 1e-5
NEG = -3.0e38
RB = 256
NB = N // RB
RBC = 128
NPC = (B * N) // RBC
CNT_E = float(B * N * KNN)
CNT_5 = float(B * N)
NW = 32
PPW = (B * N) // NW
GCH = 32
GJ = (GCH * KNN) // 80
GCPW = PPW // GCH


def _lrelu(v):
    return jnp.where(v >= 0, v, 0.2 * v)


def _apply_bn(z, st, g, b, cnt):
    mean = st[0:1, :] / cnt
    var = st[1:2, :] / cnt - mean * mean
    return (z - mean) / jnp.sqrt(var + EPS) * g + b


def _pd_common(x_full, rows, bidx, idx_ref):
    colsq = jnp.sum(x_full * x_full, axis=1)
    inner = -2.0 * lax.dot_general(
        rows.astype(BF16), x_full.astype(BF16), (((1,), (1,)), ((), ())),
        preferred_element_type=F32)
    pd = ((-jnp.sum(rows * rows, axis=1, keepdims=True)) - inner
          - colsq[None, :])
    iota = lax.broadcasted_iota(I32, (RB, N), 1)
    cur = pd
    cols = []
    for _ in range(KNN):
        m = jnp.max(cur, axis=1, keepdims=True)
        iv = jnp.min(jnp.where(cur >= m, iota, N), axis=1, keepdims=True)
        cols.append(iv)
        cur = jnp.where(iota == iv, NEG, cur)
    idx_ref[...] = jnp.concatenate(cols, axis=1) + bidx * N


def _pd_first_body(x_ref, xr_ref, idx_ref):
    _pd_common(x_ref[0], xr_ref[0], pl.program_id(0), idx_ref)


def _pd_first(xpad):
    return pl.pallas_call(
        _pd_first_body,
        grid=(B, NB),
        in_specs=[
            pl.BlockSpec((1, N, CP), lambda b, i: (b, 0, 0)),
            pl.BlockSpec((1, RB, CP), lambda b, i: (b, i, 0)),
        ],
        out_specs=pl.BlockSpec((RB, KNN), lambda b, i: (b * NB + i, 0)),
        out_shape=jax.ShapeDtypeStruct((B * N, KNN), I32),
    )(xpad.reshape(B, N, CP), xpad.reshape(B, N, CP))


def _pd_next_body(z_ref, zr_ref, st_ref, g_ref, b_ref, idx_ref, xp_ref):
    x_full = _lrelu(_apply_bn(z_ref[0], st_ref[...], g_ref[...], b_ref[...],
                              CNT_E))
    rows = _lrelu(_apply_bn(zr_ref[0], st_ref[...], g_ref[...], b_ref[...],
                            CNT_E))
    C = x_full.shape[1]
    if C < CP:
        pad = jnp.zeros((RB, CP - C), F32)
        xp_ref[...] = jnp.concatenate([rows, pad], axis=1)
    else:
        xp_ref[...] = rows
    _pd_common(x_full, rows, pl.program_id(0), idx_ref)


def _pd_next(z, st, g, b, C):
    call = pl.pallas_call(
        _pd_next_body,
        grid=(B, NB),
        in_specs=[
            pl.BlockSpec((1, N, C), lambda bb, i: (bb, 0, 0)),
            pl.BlockSpec((1, RB, C), lambda bb, i: (bb, i, 0)),
            pl.BlockSpec((2, C), lambda bb, i: (0, 0)),
            pl.BlockSpec((1, C), lambda bb, i: (0, 0)),
            pl.BlockSpec((1, C), lambda bb, i: (0, 0)),
        ],
        out_specs=[
            pl.BlockSpec((RB, KNN), lambda bb, i: (bb * NB + i, 0)),
            pl.BlockSpec((RB, CP), lambda bb, i: (bb * NB + i, 0)),
        ],
        out_shape=[
            jax.ShapeDtypeStruct((B * N, KNN), I32),
            jax.ShapeDtypeStruct((B * N, CP), F32),
        ],
    )
    zr = z.reshape(B, N, C)
    return call(zr, zr, st, g, b)


def _gather_rows(table, idx):
    idx_r = idx.reshape(NW * GCPW, GJ, 80)
    mesh = plsc.VectorSubcoreMesh(core_axis_name="c", subcore_axis_name="s")

    @functools.partial(
        pl.kernel,
        mesh=mesh,
        out_type=jax.ShapeDtypeStruct((B * N * KNN, CP), F32),
        scratch_types=[
            pltpu.VMEM((GJ, 80), I32),
            pltpu.VMEM((GCH * KNN, CP), F32),
            pltpu.SemaphoreType.DMA,
        ],
    )
    def kfn(tab_h, idx_h, out_h, idx_v, rows_v, sem):
        wid = lax.axis_index("s") * 2 + lax.axis_index("c")

        def chunk_body(c, carry):
            gch = wid * GCPW + c
            pltpu.sync_copy(idx_h.at[gch], idx_v)
            handles = [
                pltpu.async_copy(tab_h.at[idx_v.at[j]],
                                 rows_v.at[pl.ds(j * 80, 80)], sem)
                for j in range(GJ)
            ]
            for h in handles:
                h.wait()
            pltpu.sync_copy(rows_v,
                            out_h.at[pl.ds(gch * (GCH * KNN), GCH * KNN)])
            return carry

        lax.fori_loop(0, GCPW, chunk_body, 0)

    return kfn(table, idx_r)


def _conv_body(xg_ref, xi_ref, w_ref, z_ref, st_ref, acc_ref):
    j = pl.program_id(0)
    xi = xi_ref[...]
    xg = xg_ref[...]
    xi_rep = jnp.broadcast_to(xi[:, None, :], (RBC, KNN, CP))
    diff = (xg.reshape(RBC, KNN, CP) - xi_rep).reshape(RBC * KNN, CP)
    feat = jnp.concatenate([diff, xi_rep.reshape(RBC * KNN, CP)], axis=1)
    fdot = lax.dot_general(feat.astype(BF16), w_ref[...],
                           (((1,), (0,)), ((), ())),
                           preferred_element_type=F32)
    O = fdot.shape[1]
    f = fdot.reshape(RBC, KNN, O)
    z_ref[...] = jnp.max(f, axis=1)
    cur = jnp.concatenate(
        [jnp.sum(f.reshape(RBC * KNN, O), axis=0, keepdims=True),
         jnp.sum((f * f).reshape(RBC * KNN, O), axis=0, keepdims=True)],
        axis=0)

    @pl.when(j == 0)
    def _():
        acc_ref[...] = cur

    @pl.when(j > 0)
    def _():
        acc_ref[...] = acc_ref[...] + cur

    @pl.when(j == NPC - 1)
    def _():
        st_ref[...] = acc_ref[...]


def _conv(xg, xpad, wT, O):
    return pl.pallas_call(
        _conv_body,
        grid=(NPC,),
        in_specs=[
            pl.BlockSpec((RBC * KNN, CP), lambda j: (j, 0)),
            pl.BlockSpec((RBC, CP), lambda j: (j, 0)),
            pl.BlockSpec((2 * CP, O), lambda j: (0, 0)),
        ],
        out_specs=[
            pl.BlockSpec((RBC, O), lambda j: (j, 0)),
            pl.BlockSpec((2, O), lambda j: (0, 0)),
        ],
        out_shape=[
            jax.ShapeDtypeStruct((B * N, O), F32),
            jax.ShapeDtypeStruct((2, O), F32),
        ],
        scratch_shapes=[pltpu.VMEM((2, O), F32)],
    )(xg, xpad, wT)


def _f1_body(z1, s1, g1, b1, z2, s2, g2, b2, z3, s3, g3, b3, z4, s4, g4, b4,
             w5t, u_ref, st_ref, acc_ref):
    bb = pl.program_id(0)
    i = pl.program_id(1)
    xs = []
    for (zr, sr, gr, br) in ((z1, s1, g1, b1), (z2, s2, g2, b2),
                             (z3, s3, g3, b3), (z4, s4, g4, b4)):
        xs.append(_lrelu(_apply_bn(zr[...], sr[...], gr[...], br[...],
                                   CNT_E)))
    xc = jnp.concatenate(xs, axis=1)
    u = lax.dot_general(xc.astype(BF16), w5t[...], (((1,), (0,)), ((), ())),
                        preferred_element_type=F32)
    u_ref[...] = u
    cur = jnp.concatenate([jnp.sum(u, axis=0, keepdims=True),
                           jnp.sum(u * u, axis=0, keepdims=True)], axis=0)
    first = (bb == 0) & (i == 0)

    @pl.when(first)
    def _():
        acc_ref[...] = cur

    @pl.when(jnp.logical_not(first))
    def _():
        acc_ref[...] = acc_ref[...] + cur

    @pl.when((bb == B - 1) & (i == NB - 1))
    def _():
        st_ref[...] = acc_ref[...]


def _f1(z1, s1, g1, b1, z2, s2, g2, b2, z3, s3, g3, b3, z4, s4, g4, b4, w5t):
    def zspec(C):
        return pl.BlockSpec((RB, C), lambda bb, i: (bb * NB + i, 0))

    def sspec(C):
        return pl.BlockSpec((2, C), lambda bb, i: (0, 0))

    def vspec(C):
        return pl.BlockSpec((1, C), lambda bb, i: (0, 0))

    return pl.pallas_call(
        _f1_body,
        grid=(B, NB),
        in_specs=[
            zspec(64), sspec(64), vspec(64), vspec(64),
            zspec(64), sspec(64), vspec(64), vspec(64),
            zspec(128), sspec(128), vspec(128), vspec(128),
            zspec(256), sspec(256), vspec(256), vspec(256),
            pl.BlockSpec((512, 1024), lambda bb, i: (0, 0)),
        ],
        out_specs=[
            pl.BlockSpec((RB, 1024), lambda bb, i: (bb * NB + i, 0)),
            pl.BlockSpec((2, 1024), lambda bb, i: (0, 0)),
        ],
        out_shape=[
            jax.ShapeDtypeStruct((B * N, 1024), F32),
            jax.ShapeDtypeStruct((2, 1024), F32),
        ],
        scratch_shapes=[pltpu.VMEM((2, 1024), F32)],
    )(z1, s1, g1, b1, z2, s2, g2, b2, z3, s3, g3, b3, z4, s4, g4, b4, w5t)


NB2 = 16
NBLK = N // NB2


def _bn_head(t, g, b):
    m = jnp.mean(t, axis=0, keepdims=True)
    v = jnp.mean(t * t, axis=0, keepdims=True) - m * m
    return (t - m) / jnp.sqrt(v + EPS) * g + b


def _f2_body(u_ref, st_ref, g5, b5, w6t, g6, b6, w7t, bias7, g7, b7, w8t,
             bias8, out_ref, rmax_ref, rsum_ref):
    j = pl.program_id(0)
    mean = st_ref[0:1, :] / CNT_5
    var = st_ref[1:2, :] / CNT_5 - mean * mean
    v = _lrelu((u_ref[...] - mean[None]) / jnp.sqrt(var + EPS)[None]
               * g5[...][None] + b5[...][None])
    pm = jnp.max(v, axis=1)
    ps = jnp.sum(v, axis=1)

    @pl.when(j == 0)
    def _():
        rmax_ref[...] = pm
        rsum_ref[...] = ps

    @pl.when(j > 0)
    def _():
        rmax_ref[...] = jnp.maximum(rmax_ref[...], pm)
        rsum_ref[...] = rsum_ref[...] + ps

    @pl.when(j == NB2 - 1)
    def _():
        h = jnp.concatenate([rmax_ref[...], rsum_ref[...] / float(N)], axis=1)
        t = lax.dot_general(h.astype(BF16), w6t[...], (((1,), (0,)), ((), ())),
                            preferred_element_type=F32)
        t = _lrelu(_bn_head(t, g6[...], b6[...]))
        t = lax.dot_general(t.astype(BF16), w7t[...], (((1,), (0,)), ((), ())),
                            preferred_element_type=F32) + bias7[...]
        t = _lrelu(_bn_head(t, g7[...], b7[...]))
        t = lax.dot_general(t.astype(BF16), w8t[...], (((1,), (0,)), ((), ())),
                            preferred_element_type=F32) + bias8[...]
        out_ref[...] = t


def _f2(u3, st, g5, b5, w6t, g6, b6, w7t, bias7, g7, b7, w8tp, bias8p):
    def vspec(C):
        return pl.BlockSpec((1, C), lambda j: (0, 0))

    return pl.pallas_call(
        _f2_body,
        grid=(NB2,),
        in_specs=[
            pl.BlockSpec((B, NBLK, 1024), lambda j: (0, j, 0)),
            pl.BlockSpec((2, 1024), lambda j: (0, 0)),
            vspec(1024), vspec(1024),
            pl.BlockSpec((2048, 512), lambda j: (0, 0)),
            vspec(512), vspec(512),
            pl.BlockSpec((512, 256), lambda j: (0, 0)),
            vspec(256), vspec(256), vspec(256),
            pl.BlockSpec((256, 8), lambda j: (0, 0)),
            vspec(8),
        ],
        out_specs=pl.BlockSpec((B, 8), lambda j: (0, 0)),
        out_shape=jax.ShapeDtypeStruct((B, 8), F32),
        scratch_shapes=[pltpu.VMEM((B, 1024), F32),
                        pltpu.VMEM((B, 1024), F32)],
    )(u3, st, g5, b5, w6t, g6, b6, w7t, bias7, g7, b7, w8tp, bias8p)


def _edge_layer(xpad, idx, W, C, O):
    xg = _gather_rows(xpad, idx)
    wa = jnp.pad(W[:, :C].T, ((0, CP - C), (0, 0)))
    wb = jnp.pad(W[:, C:].T, ((0, CP - C), (0, 0)))
    wT = jnp.concatenate([wa, wb], axis=0).astype(BF16)
    return _conv(xg, xpad, wT, O)


def kernel(x, W1, g1, b1, W2, g2, b2, W3, g3, b3, W4, g4, b4, W5, g5, b5,
           W6, g6, b6, W7, bias7, g7, b7, W8, bias8):
    xpad1 = jnp.pad(x, ((0, 0), (0, 0), (0, CP - 3))).reshape(B * N, CP)

    idx1 = _pd_first(xpad1)
    z1, st1 = _edge_layer(xpad1, idx1, W1, 3, 64)

    idx2, xpad2 = _pd_next(z1, st1, g1.reshape(1, 64), b1.reshape(1, 64), 64)
    z2, st2 = _edge_layer(xpad2, idx2, W2, 64, 64)

    idx3, xpad3 = _pd_next(z2, st2, g2.reshape(1, 64), b2.reshape(1, 64), 64)
    z3, st3 = _edge_layer(xpad3, idx3, W3, 64, 128)

    idx4, xpad4 = _pd_next(z3, st3, g3.reshape(1, 128), b3.reshape(1, 128),
                           128)
    z4, st4 = _edge_layer(xpad4, idx4, W4, 128, 256)

    u, st5 = _f1(z1, st1, g1.reshape(1, 64), b1.reshape(1, 64),
                 z2, st2, g2.reshape(1, 64), b2.reshape(1, 64),
                 z3, st3, g3.reshape(1, 128), b3.reshape(1, 128),
                 z4, st4, g4.reshape(1, 256), b4.reshape(1, 256),
                 W5.T.astype(BF16))

    out = _f2(u.reshape(B, N, 1024), st5, g5.reshape(1, 1024),
              b5.reshape(1, 1024), W6.T.astype(BF16), g6.reshape(1, 512),
              b6.reshape(1, 512), W7.T.astype(BF16), bias7.reshape(1, 256),
              g7.reshape(1, 256), b7.reshape(1, 256),
              jnp.pad(W8.T, ((0, 0), (0, 5))).astype(BF16),
              jnp.pad(bias8, (0, 5)).reshape(1, 8))
    return out[:, :3]
